# Initial kernel scaffold; baseline (speedup 1.0000x reference)
import jax
import jax.numpy as jnp
from jax import lax
from jax.experimental import pallas as pl

N = 32768
D_NODE = 128
D_HID = 1024
K = 64


def _body(nodes_ref, hid_ref, w_ref, b_ref, out_ref):
    nodes64 = nodes_ref[...]
    hid = hid_ref[...]
    W = w_ref[...]
    b = b_ref[...]

    f32 = jnp.float32
    h = lax.dot_general(hid, W, (((1,), (1,)), ((), ())),
                        preferred_element_type=f32) + b
    s_row = lax.dot_general(h, nodes64, (((1,), (1,)), ((), ())),
                            preferred_element_type=f32)

    ones_mat = jnp.ones((K, K), f32)
    I = lax.broadcasted_iota(jnp.int32, (K, K), 0)
    J = lax.broadcasted_iota(jnp.int32, (K, K), 1)
    ident = (I == J).astype(f32)

    S2 = jnp.broadcast_to(s_row, (K, K))
    S1 = lax.dot_general(ident * S2, ones_mat, (((1,), (0,)), ((), ())),
                         preferred_element_type=f32)

    C = ((S1 < S2) | ((S1 == S2) & (I < J))).astype(f32)
    rank_row = lax.dot_general(jnp.ones((1, K), f32), C,
                               (((1,), (0,)), ((), ())),
                               preferred_element_type=f32)
    rank_mat = jnp.broadcast_to(rank_row, (K, K))
    P = (rank_mat == lax.broadcasted_iota(f32, (K, K), 0)).astype(f32)
    out_ref[...] = lax.dot_general(P, nodes64, (((1,), (0,)), ((), ())),
                                   preferred_element_type=f32)


def kernel(nodes, hidden_state, W, b):
    out = pl.pallas_call(
        _body,
        in_specs=[
            pl.BlockSpec((K, D_NODE), lambda: (0, 0)),
            pl.BlockSpec((1, D_HID), lambda: (0, 0)),
            pl.BlockSpec((D_NODE, D_HID), lambda: (0, 0)),
            pl.BlockSpec((1, D_NODE), lambda: (0, 0)),
        ],
        out_specs=pl.BlockSpec((K, D_NODE), lambda: (0, 0)),
        out_shape=jax.ShapeDtypeStruct((K, D_NODE), jnp.float32),
    )(nodes, hidden_state, W, b.reshape(1, D_NODE))
    return out.reshape(1, K * D_NODE)


# speedup vs baseline: 19.4101x; 19.4101x over previous
import jax
import jax.numpy as jnp
from jax import lax
from jax.experimental import pallas as pl

N = 32768
D_NODE = 128
D_HID = 1024
K = 64


def _body(nodes_ref, hid_ref, w_ref, b_ref, out_ref):
    nodes64 = nodes_ref[...]
    hid = hid_ref[...]
    W = w_ref[...]
    b = b_ref[...]

    f32 = jnp.float32
    hi = lax.Precision.HIGHEST
    h = lax.dot_general(hid, W, (((1,), (1,)), ((), ())),
                        preferred_element_type=f32) + b
    s_row = lax.dot_general(h, nodes64, (((1,), (1,)), ((), ())),
                            preferred_element_type=f32)

    I = lax.broadcasted_iota(jnp.int32, (K, K), 0)
    J = lax.broadcasted_iota(jnp.int32, (K, K), 1)

    s_col = jnp.transpose(s_row, (1, 0))
    S1 = jnp.broadcast_to(s_col, (K, K))
    S2 = jnp.broadcast_to(s_row, (K, K))

    C = ((S1 < S2) | ((S1 == S2) & (I < J))).astype(f32)
    rank_row = jnp.sum(C, axis=0, keepdims=True)
    rank_mat = jnp.broadcast_to(rank_row, (K, K)).astype(jnp.int32)
    P = (rank_mat == I).astype(f32)
    out_ref[...] = lax.dot_general(P, nodes64, (((1,), (0,)), ((), ())),
                                   preferred_element_type=f32, precision=hi)


def kernel(nodes, hidden_state, W, b):
    out = pl.pallas_call(
        _body,
        grid=(1,),
        in_specs=[
            pl.BlockSpec((K, D_NODE), lambda i: (0, 0)),
            pl.BlockSpec((1, D_HID), lambda i: (0, 0)),
            pl.BlockSpec((D_NODE, D_HID), lambda i: (0, 0)),
            pl.BlockSpec((1, D_NODE), lambda i: (0, 0)),
        ],
        out_specs=pl.BlockSpec((K, D_NODE), lambda i: (0, 0)),
        out_shape=jax.ShapeDtypeStruct((K, D_NODE), jnp.float32),
    )(nodes, hidden_state, W, b.reshape(1, D_NODE))
    return out.reshape(1, K * D_NODE)
